# baseline (device time: 49016 ns/iter reference)
import jax
import jax.numpy as jnp
from jax import lax
from jax.experimental import pallas as pl
from jax.experimental.pallas import tpu as pltpu

C = 8

_MESH = pl.DeviceIdType.MESH


def kernel(x):
    m, n = x.shape
    q = m // 4
    ch = q // C

    g_out = 2 * lax.axis_index("x") + lax.axis_index("z")
    x_q = lax.dynamic_slice_in_dim(x, g_out * q, q, axis=0).astype(
        jnp.bfloat16
    )

    def body(xq_ref, out_ref, rp1_ref, r_ref,
             p1_send, p1_recv, px_send, px_recv, pz_send, pz_recv,
             fy_send, fy_recv, fx_send, fx_recv, fz_send, fz_recv):
        my_x = lax.axis_index("x")
        my_y = lax.axis_index("y")
        my_z = lax.axis_index("z")
        y_partner = (my_x, 1 - my_y, my_z)
        x_nbr = (1 - my_x, my_y, my_z)
        z_nbr = (my_x, my_y, 1 - my_z)
        g = 2 * my_x + my_z
        g_x = 2 * (1 - my_x) + my_z
        g_z = 2 * my_x + (1 - my_z)
        g_d = 2 * (1 - my_x) + (1 - my_z)

        def rc(region, c):
            return out_ref.at[pl.ds(region * q + c * ch, ch)]

        barrier_sem = pltpu.get_barrier_semaphore()
        for nbr in (y_partner, x_nbr, z_nbr):
            pl.semaphore_signal(
                barrier_sem, inc=1, device_id=nbr, device_id_type=_MESH,
            )
        pl.semaphore_wait(barrier_sem, 3)

        p1 = []
        for c in range(C):
            ds = pl.ds(c * ch, ch)
            d = pltpu.make_async_remote_copy(
                src_ref=xq_ref.at[ds], dst_ref=rp1_ref.at[ds],
                send_sem=p1_send.at[c], recv_sem=p1_recv.at[c],
                device_id=y_partner, device_id_type=_MESH,
            )
            d.start()
            p1.append(d)

        FZ = (0, 1, 5, 6, 7)

        px = []
        for c in range(C):
            ds = pl.ds(c * ch, ch)
            p1[c].wait_recv()
            r_ref[ds] = xq_ref[ds] + rp1_ref[ds]
            d = pltpu.make_async_remote_copy(
                src_ref=r_ref.at[ds], dst_ref=rc(g, c),
                send_sem=px_send.at[c], recv_sem=px_recv.at[c],
                device_id=x_nbr, device_id_type=_MESH,
            )
            d.start()
            px.append(d)
            out_ref[pl.ds(g * q + c * ch, ch)] = r_ref[ds]

            if c < 6:
                @pl.when(my_y == 0)
                def _(c=c, ds=ds):
                    dd = pltpu.make_async_remote_copy(
                        src_ref=r_ref.at[ds], dst_ref=rc(g, c),
                        send_sem=pz_send.at[c], recv_sem=pz_recv.at[c],
                        device_id=z_nbr, device_id_type=_MESH,
                    )
                    dd.start()
            if c >= 2:
                @pl.when(my_y == 1)
                def _(c=c, ds=ds):
                    dd = pltpu.make_async_remote_copy(
                        src_ref=r_ref.at[ds], dst_ref=rc(g, c),
                        send_sem=pz_send.at[c - 2], recv_sem=pz_recv.at[c - 2],
                        device_id=z_nbr, device_id_type=_MESH,
                    )
                    dd.start()

            if c in (1, 2):
                k = c - 1
                px[k].wait_recv()
                dd = pltpu.make_async_remote_copy(
                    src_ref=rc(g_x, k), dst_ref=rc(g_x, k),
                    send_sem=fz_send.at[k], recv_sem=fz_recv.at[k],
                    device_id=z_nbr, device_id_type=_MESH,
                )
                dd.start()

        dummy_src = r_ref.at[pl.ds(0, ch)]

        @pl.when(my_y == 0)
        def _():
            for j in range(6):
                c = j
                w = pltpu.make_async_remote_copy(
                    src_ref=dummy_src, dst_ref=rc(g_z, c),
                    send_sem=pz_send.at[j], recv_sem=pz_recv.at[j],
                    device_id=z_nbr, device_id_type=_MESH,
                )
                w.wait_recv()
                if c in (0, 1):
                    dd = pltpu.make_async_remote_copy(
                        src_ref=rc(g_z, c), dst_ref=rc(g_z, c),
                        send_sem=fy_send.at[c], recv_sem=fy_recv.at[c],
                        device_id=y_partner, device_id_type=_MESH,
                    )
                    dd.start()
                if c in (2, 3, 4):
                    dd = pltpu.make_async_remote_copy(
                        src_ref=rc(g_z, c), dst_ref=rc(g_z, c),
                        send_sem=fx_send.at[c - 2], recv_sem=fx_recv.at[c - 2],
                        device_id=x_nbr, device_id_type=_MESH,
                    )
                    dd.start()

        @pl.when(my_y == 1)
        def _():
            for j in range(6):
                c = j + 2
                w = pltpu.make_async_remote_copy(
                    src_ref=dummy_src, dst_ref=rc(g_z, c),
                    send_sem=pz_send.at[j], recv_sem=pz_recv.at[j],
                    device_id=z_nbr, device_id_type=_MESH,
                )
                w.wait_recv()
                if c in (2, 3, 4):
                    dd = pltpu.make_async_remote_copy(
                        src_ref=rc(g_z, c), dst_ref=rc(g_z, c),
                        send_sem=fx_send.at[c - 2], recv_sem=fx_recv.at[c - 2],
                        device_id=x_nbr, device_id_type=_MESH,
                    )
                    dd.start()
                if c in (6, 7):
                    dd = pltpu.make_async_remote_copy(
                        src_ref=rc(g_z, c), dst_ref=rc(g_z, c),
                        send_sem=fy_send.at[c - 6], recv_sem=fy_recv.at[c - 6],
                        device_id=y_partner, device_id_type=_MESH,
                    )
                    dd.start()

        for k, c in ((2, 5), (3, 6), (4, 7)):
            px[c].wait_recv()
            dd = pltpu.make_async_remote_copy(
                src_ref=rc(g_x, c), dst_ref=rc(g_x, c),
                send_sem=fz_send.at[k], recv_sem=fz_recv.at[k],
                device_id=z_nbr, device_id_type=_MESH,
            )
            dd.start()

        for c in (2, 3, 4):
            px[c].wait_recv()
        for j in range(2):
            cv = 6 * (1 - my_y) + j
            pltpu.make_async_remote_copy(
                src_ref=dummy_src,
                dst_ref=out_ref.at[pl.ds(g_z * q + cv * ch, ch)],
                send_sem=fy_send.at[j], recv_sem=fy_recv.at[j],
                device_id=y_partner, device_id_type=_MESH,
            ).wait_recv()
        for k in range(3):
            pltpu.make_async_remote_copy(
                src_ref=dummy_src, dst_ref=rc(g_d, 2 + k),
                send_sem=fx_send.at[k], recv_sem=fx_recv.at[k],
                device_id=x_nbr, device_id_type=_MESH,
            ).wait_recv()
        for k in range(5):
            pltpu.make_async_remote_copy(
                src_ref=dummy_src, dst_ref=rc(g_d, FZ[k]),
                send_sem=fz_send.at[k], recv_sem=fz_recv.at[k],
                device_id=z_nbr, device_id_type=_MESH,
            ).wait_recv()

        for d in p1 + px:
            d.wait_send()
        for sems, cnt, dev in (
            (pz_send, 6, z_nbr),
            (fy_send, 2, y_partner),
            (fx_send, 3, x_nbr),
            (fz_send, 5, z_nbr),
        ):
            for j in range(cnt):
                pltpu.make_async_remote_copy(
                    src_ref=dummy_src, dst_ref=rc(g, 0),
                    send_sem=sems.at[j], recv_sem=pz_recv.at[0],
                    device_id=dev, device_id_type=_MESH,
                ).wait_send()

    return pl.pallas_call(
        body,
        out_shape=jax.ShapeDtypeStruct((m, n), jnp.bfloat16),
        in_specs=[pl.BlockSpec(memory_space=pltpu.VMEM)],
        out_specs=pl.BlockSpec(memory_space=pltpu.VMEM),
        scratch_shapes=[
            pltpu.VMEM((q, n), jnp.bfloat16),
            pltpu.VMEM((q, n), jnp.bfloat16),
            pltpu.SemaphoreType.DMA((C,)),
            pltpu.SemaphoreType.DMA((C,)),
            pltpu.SemaphoreType.DMA((C,)),
            pltpu.SemaphoreType.DMA((C,)),
            pltpu.SemaphoreType.DMA((6,)),
            pltpu.SemaphoreType.DMA((6,)),
            pltpu.SemaphoreType.DMA((2,)),
            pltpu.SemaphoreType.DMA((2,)),
            pltpu.SemaphoreType.DMA((3,)),
            pltpu.SemaphoreType.DMA((3,)),
            pltpu.SemaphoreType.DMA((5,)),
            pltpu.SemaphoreType.DMA((5,)),
        ],
        compiler_params=pltpu.CompilerParams(collective_id=0),
    )(x_q)
